# baseline (speedup 1.0000x reference)
import functools

import jax
import jax.numpy as jnp
from jax import lax
from jax.experimental import pallas as pl
from jax.experimental.pallas import tpu as pltpu
from jax.experimental.pallas import tpu_sc as plsc

_E = 8
_K = 2
_D = 1024
_H = 512
_T = 2048
_S = _T * _K

_BLK = 128
_NB = _S // _BLK + _E
_P = _NB * _BLK

_NC = 2
_NS = 16
_NW = _NC * _NS

_GATHER_CHUNK = 80
_COMBINE_CHUNK = 32


def _routing_metadata(indices, weights):
    e_flat = indices.reshape(-1).astype(jnp.int32)
    t_flat = jnp.repeat(jnp.arange(_T, dtype=jnp.int32), _K)
    w_flat = weights.reshape(-1).astype(jnp.float32)

    onehot = (e_flat[:, None] == jnp.arange(_E, dtype=jnp.int32)[None, :])
    incl = jnp.cumsum(onehot.astype(jnp.int32), axis=0)
    counts = incl[-1]
    rwe = jnp.take_along_axis(incl, e_flat[:, None], axis=1)[:, 0] - 1

    nblk = (counts + _BLK - 1) // _BLK
    po_blk = jnp.concatenate(
        [jnp.zeros((1,), jnp.int32), jnp.cumsum(nblk)[:-1].astype(jnp.int32)])
    p_flat = po_blk[e_flat] * _BLK + rwe

    g_tok = jnp.zeros((_P,), jnp.int32).at[p_flat].set(t_flat)
    g_w = jnp.zeros((_P,), jnp.float32).at[p_flat].set(w_flat)

    blk = jnp.arange(_NB, dtype=jnp.int32)
    eid = jnp.clip(jnp.searchsorted(po_blk, blk, side="right") - 1, 0, _E - 1)
    eid = eid.astype(jnp.int32)

    p_pair = p_flat.reshape(_T, _K)
    return g_tok, g_w, eid, p_pair[:, 0], p_pair[:, 1]


def _sc_gather_rows(x, idx):
    per_w = _P // _NW
    nchunk = per_w // _GATHER_CHUNK
    mesh = plsc.VectorSubcoreMesh(core_axis_name="c", subcore_axis_name="s")

    @functools.partial(
        pl.kernel,
        mesh=mesh,
        out_type=jax.ShapeDtypeStruct((_P, _D), jnp.float32),
        scratch_types=[
            pltpu.VMEM((_GATHER_CHUNK,), jnp.int32),
            pltpu.VMEM((_GATHER_CHUNK, _D), jnp.float32),
            pltpu.SemaphoreType.DMA,
        ],
    )
    def k(x_hbm, idx_hbm, out_hbm, idx_v, rows_v, sem):
        wid = lax.axis_index("s") * _NC + lax.axis_index("c")
        base = wid * per_w

        def body(c, carry):
            start = base + c * _GATHER_CHUNK
            pltpu.sync_copy(idx_hbm.at[pl.ds(start, _GATHER_CHUNK)], idx_v)
            pltpu.async_copy(x_hbm.at[idx_v], rows_v, sem).wait()
            pltpu.sync_copy(rows_v, out_hbm.at[pl.ds(start, _GATHER_CHUNK)])
            return carry

        lax.fori_loop(0, nchunk, body, 0)

    return k(x, idx)


def _sc_combine(og, p0, p1):
    per_w = _T // _NW
    nchunk = per_w // _COMBINE_CHUNK
    mesh = plsc.VectorSubcoreMesh(core_axis_name="c", subcore_axis_name="s")

    @functools.partial(
        pl.kernel,
        mesh=mesh,
        out_type=jax.ShapeDtypeStruct((_T, _D), jnp.float32),
        scratch_types=[
            pltpu.VMEM((_COMBINE_CHUNK,), jnp.int32),
            pltpu.VMEM((_COMBINE_CHUNK,), jnp.int32),
            pltpu.VMEM((_COMBINE_CHUNK, _D), jnp.float32),
            pltpu.VMEM((_COMBINE_CHUNK, _D), jnp.float32),
            pltpu.SemaphoreType.DMA,
        ],
    )
    def k(og_hbm, p0_hbm, p1_hbm, y_hbm, i0_v, i1_v, a_v, b_v, sem):
        wid = lax.axis_index("s") * _NC + lax.axis_index("c")
        base = wid * per_w

        def chunk_body(c, carry):
            start = base + c * _COMBINE_CHUNK
            pltpu.sync_copy(p0_hbm.at[pl.ds(start, _COMBINE_CHUNK)], i0_v)
            pltpu.sync_copy(p1_hbm.at[pl.ds(start, _COMBINE_CHUNK)], i1_v)
            pltpu.async_copy(og_hbm.at[i0_v], a_v, sem).wait()
            pltpu.async_copy(og_hbm.at[i1_v], b_v, sem).wait()

            def row_body(r, rc):
                for j in range(_D // 16):
                    sl = pl.ds(j * 16, 16)
                    a_v[r, sl] = a_v[r, sl] + b_v[r, sl]
                return rc

            lax.fori_loop(0, _COMBINE_CHUNK, row_body, 0)
            pltpu.sync_copy(a_v, y_hbm.at[pl.ds(start, _COMBINE_CHUNK)])
            return carry

        lax.fori_loop(0, nchunk, chunk_body, 0)

    return k(og, p0, p1)


def _tc_grouped_mlp(xg, W1, W2, wp, eid):
    def body(eid_ref, xg_ref, w1_ref, w2_ref, wp_ref, out_ref):
        xb = xg_ref[...]
        w1 = w1_ref[0]
        h = lax.dot_general(xb, w1, (((1,), (1,)), ((), ())),
                            preferred_element_type=jnp.float32)
        gate = h[:, :_H]
        up = h[:, _H:]
        a = gate * jax.nn.sigmoid(gate) * up
        w2 = w2_ref[0]
        out = lax.dot_general(a, w2, (((1,), (1,)), ((), ())),
                              preferred_element_type=jnp.float32)
        out_ref[...] = out * wp_ref[...]

    grid_spec = pltpu.PrefetchScalarGridSpec(
        num_scalar_prefetch=1,
        grid=(_NB,),
        in_specs=[
            pl.BlockSpec((_BLK, _D), lambda i, eid: (i, 0)),
            pl.BlockSpec((1, 2 * _H, _D), lambda i, eid: (eid[i], 0, 0)),
            pl.BlockSpec((1, _D, _H), lambda i, eid: (eid[i], 0, 0)),
            pl.BlockSpec((_BLK, 1), lambda i, eid: (i, 0)),
        ],
        out_specs=pl.BlockSpec((_BLK, _D), lambda i, eid: (i, 0)),
    )
    return pl.pallas_call(
        body,
        grid_spec=grid_spec,
        out_shape=jax.ShapeDtypeStruct((_P, _D), jnp.float32),
        compiler_params=pltpu.CompilerParams(
            dimension_semantics=("arbitrary",)),
    )(eid, xg, W1, W2, wp)


def kernel(x, weights, indices, W1, W2):
    g_tok, g_w, eid, p0, p1 = _routing_metadata(indices, weights)
    xg = _sc_gather_rows(x, g_tok)
    og = _tc_grouped_mlp(xg, W1, W2, g_w[:, None], eid)
    y = _sc_combine(og, p0, p1)
    return y


# speedup vs baseline: 1.6167x; 1.6167x over previous
import functools

import jax
import jax.numpy as jnp
from jax import lax
from jax.experimental import pallas as pl
from jax.experimental.pallas import tpu as pltpu
from jax.experimental.pallas import tpu_sc as plsc

_E = 8
_K = 2
_D = 1024
_H = 512
_T = 2048
_S = _T * _K

_BLK = 128
_NB = _S // _BLK + _E
_P = _NB * _BLK

_NC = 2
_NS = 16
_NW = _NC * _NS

_DCH = 32
_DNC = _S // _NW // _DCH
_CCH = 16
_CNC = _T // _NW // _CCH


def _routing_metadata(indices):
    e_flat = indices.reshape(-1).astype(jnp.int32)
    onehot = (e_flat[:, None] == jnp.arange(_E, dtype=jnp.int32)[None, :])
    onehot = onehot.astype(jnp.int32)
    incl = jnp.cumsum(onehot, axis=0)
    rwe = jnp.sum(incl * onehot, axis=1) - 1
    counts = incl[-1]
    nblk = (counts + _BLK - 1) // _BLK
    po_blk = jnp.concatenate(
        [jnp.zeros((1,), jnp.int32), jnp.cumsum(nblk)[:-1].astype(jnp.int32)])
    p_flat = jnp.sum(onehot * po_blk[None, :], axis=1) * _BLK + rwe

    blk = jnp.arange(_NB, dtype=jnp.int32)
    eid = jnp.sum((po_blk[None, :] <= blk[:, None]).astype(jnp.int32), axis=1) - 1
    eid = jnp.clip(eid, 0, _E - 1).astype(jnp.int32)
    return p_flat, eid


def _sc_dispatch(x, t_flat, p3):
    per_w = _S // _NW
    mesh = plsc.VectorSubcoreMesh(core_axis_name="c", subcore_axis_name="s")

    @functools.partial(
        pl.kernel,
        mesh=mesh,
        out_type=jax.ShapeDtypeStruct((_P, _D), jnp.float32),
        scratch_types=[
            pltpu.VMEM((per_w,), jnp.int32),
            pltpu.VMEM((_DNC, _DCH), jnp.int32),
            pltpu.VMEM((_DCH, _D), jnp.float32),
            pltpu.VMEM((_DCH, _D), jnp.float32),
            pltpu.SemaphoreType.DMA,
            pltpu.SemaphoreType.DMA,
            pltpu.SemaphoreType.DMA,
            pltpu.SemaphoreType.DMA,
        ],
    )
    def k(x_hbm, t_hbm, p3_hbm, xg_hbm,
          t_v, p2_v, rows0, rows1, g0, g1, s0, s1):
        wid = lax.axis_index("s") * _NC + lax.axis_index("c")
        base = wid * per_w
        pltpu.sync_copy(t_hbm.at[pl.ds(base, per_w)], t_v)
        pltpu.sync_copy(p3_hbm.at[wid], p2_v)

        rows = (rows0, rows1)
        gsem = (g0, g1)
        ssem = (s0, s1)

        gh = [None] * _DNC
        sh = [None] * _DNC
        gh[0] = pltpu.async_copy(
            x_hbm.at[t_v.at[pl.ds(0, _DCH)]], rows[0], gsem[0])
        for c in range(_DNC):
            k_ = c % 2
            gh[c].wait()
            sh[c] = pltpu.async_copy(rows[k_], xg_hbm.at[p2_v.at[c]], ssem[k_])
            if c + 1 < _DNC:
                if c >= 1:
                    sh[c - 1].wait()
                gh[c + 1] = pltpu.async_copy(
                    x_hbm.at[t_v.at[pl.ds((c + 1) * _DCH, _DCH)]],
                    rows[(c + 1) % 2], gsem[(c + 1) % 2])
        sh[_DNC - 2].wait()
        sh[_DNC - 1].wait()

    return k(x, t_flat, p3)


def _sc_combine(og, p0, p1):
    per_w = _T // _NW
    mesh = plsc.VectorSubcoreMesh(core_axis_name="c", subcore_axis_name="s")

    @functools.partial(
        pl.kernel,
        mesh=mesh,
        out_type=jax.ShapeDtypeStruct((_T, _D), jnp.float32),
        scratch_types=[
            pltpu.VMEM((per_w,), jnp.int32),
            pltpu.VMEM((per_w,), jnp.int32),
            pltpu.VMEM((_CCH, _D), jnp.float32),
            pltpu.VMEM((_CCH, _D), jnp.float32),
            pltpu.VMEM((_CCH, _D), jnp.float32),
            pltpu.VMEM((_CCH, _D), jnp.float32),
            pltpu.SemaphoreType.DMA,
            pltpu.SemaphoreType.DMA,
            pltpu.SemaphoreType.DMA,
            pltpu.SemaphoreType.DMA,
        ],
    )
    def k(og_hbm, p0_hbm, p1_hbm, y_hbm, i0_v, i1_v, a0, a1, b0, b1,
          g0, g1, s0, s1):
        wid = lax.axis_index("s") * _NC + lax.axis_index("c")
        base = wid * per_w
        pltpu.sync_copy(p0_hbm.at[pl.ds(base, per_w)], i0_v)
        pltpu.sync_copy(p1_hbm.at[pl.ds(base, per_w)], i1_v)

        a = (a0, a1)
        b = (b0, b1)
        gsem = (g0, g1)
        ssem = (s0, s1)

        def start_gathers(c):
            k_ = c % 2
            sl = pl.ds(c * _CCH, _CCH)
            ha = pltpu.async_copy(og_hbm.at[i0_v.at[sl]], a[k_], gsem[k_])
            hb = pltpu.async_copy(og_hbm.at[i1_v.at[sl]], b[k_], gsem[k_])
            return ha, hb

        gh = [None] * _CNC
        sh = [None] * _CNC
        gh[0] = start_gathers(0)
        for c in range(_CNC):
            k_ = c % 2
            gh[c][0].wait()
            gh[c][1].wait()
            if c + 1 < _CNC:
                if c >= 1:
                    sh[c - 1].wait()
                gh[c + 1] = start_gathers(c + 1)

            def row_body(r, carry, a_v=a[k_], b_v=b[k_]):
                for j in range(_D // 16):
                    sl = pl.ds(j * 16, 16)
                    a_v[r, sl] = a_v[r, sl] + b_v[r, sl]
                return carry

            lax.fori_loop(0, _CCH, row_body, 0)
            sh[c] = pltpu.async_copy(
                a[k_], y_hbm.at[pl.ds(base + c * _CCH, _CCH)], ssem[k_])
        sh[_CNC - 2].wait()
        sh[_CNC - 1].wait()

    return k(og, p0, p1)


def _tc_grouped_mlp(xg, W1, W2, wp, eid):
    def body(eid_ref, xg_ref, w1_ref, w2_ref, wp_ref, out_ref):
        xb = xg_ref[...]
        w1 = w1_ref[0]
        h = lax.dot_general(xb, w1, (((1,), (1,)), ((), ())),
                            preferred_element_type=jnp.float32)
        gate = h[:, :_H]
        up = h[:, _H:]
        a = gate * jax.nn.sigmoid(gate) * up
        w2 = w2_ref[0]
        out = lax.dot_general(a, w2, (((1,), (1,)), ((), ())),
                              preferred_element_type=jnp.float32)
        out_ref[...] = out * wp_ref[...]

    grid_spec = pltpu.PrefetchScalarGridSpec(
        num_scalar_prefetch=1,
        grid=(_NB,),
        in_specs=[
            pl.BlockSpec((_BLK, _D), lambda i, eid: (i, 0)),
            pl.BlockSpec((1, 2 * _H, _D), lambda i, eid: (eid[i], 0, 0)),
            pl.BlockSpec((1, _D, _H), lambda i, eid: (eid[i], 0, 0)),
            pl.BlockSpec((_BLK, 1), lambda i, eid: (i, 0)),
        ],
        out_specs=pl.BlockSpec((_BLK, _D), lambda i, eid: (i, 0)),
    )
    return pl.pallas_call(
        body,
        grid_spec=grid_spec,
        out_shape=jax.ShapeDtypeStruct((_P, _D), jnp.float32),
        compiler_params=pltpu.CompilerParams(
            dimension_semantics=("arbitrary",)),
    )(eid, xg, W1, W2, wp)


def kernel(x, weights, indices, W1, W2):
    p_flat, eid = _routing_metadata(indices)
    t_flat = jnp.arange(_S, dtype=jnp.int32) // _K
    w_flat = weights.reshape(-1).astype(jnp.float32)
    gw = jnp.zeros((_P,), jnp.float32).at[p_flat].set(w_flat)
    p3 = p_flat.reshape(_NW, _DNC, _DCH)
    xg = _sc_dispatch(x, t_flat, p3)
    og = _tc_grouped_mlp(xg, W1, W2, gw[:, None], eid)
    pp = p_flat.reshape(_T, _K)
    y = _sc_combine(og, pp[:, 0], pp[:, 1])
    return y
